# Initial kernel scaffold; baseline (speedup 1.0000x reference)
import functools

import jax
import jax.numpy as jnp
from jax.experimental import pallas as pl
from jax.experimental.pallas import tpu as pltpu

_VMEM_LIMIT = 48 * 1024 * 1024


def _conv_kernel(x_ref, w_ref, b_ref, o_ref, *, C, Ho, Wo):
    M = Ho * Wo
    xb = x_ref[0].astype(jnp.bfloat16)
    x5 = xb.reshape(C, Ho, 2, Wo, 2)

    p = [[x5[:, :, rh, :, rw].reshape(C, M) for rw in (0, 1)] for rh in (0, 1)]

    rmap = ((1, -1), (0, 0), (1, 0))

    col0 = (jax.lax.broadcasted_iota(jnp.int32, (1, M), 1) % Wo) == 0

    def tap(rh, sr, rw, sc):
        q = p[rh][rw]
        k = (-sr) * Wo + (-sc)
        if k:
            q = jnp.concatenate(
                [jnp.zeros((C, k), q.dtype), q[:, :M - k]], axis=1)
        if sc:
            q = jnp.where(col0, jnp.bfloat16(0), q)
        return q

    acc = jnp.broadcast_to(b_ref[...], (C, M))
    for kh in range(3):
        rh, sr = rmap[kh]
        for kw in range(3):
            rw, sc = rmap[kw]
            acc = acc + jnp.dot(w_ref[kh * 3 + kw], tap(rh, sr, rw, sc),
                                preferred_element_type=jnp.float32)
    o_ref[0] = acc.astype(o_ref.dtype)


def kernel(x, weight, bias):
    N, C, H, W = x.shape
    Ho, Wo = H // 2, W // 2
    xf = x.reshape(N, C, H * W)
    wT = jnp.transpose(weight, (0, 1, 3, 2)).reshape(9, C, C).astype(jnp.bfloat16)
    b2 = bias.astype(jnp.float32).reshape(C, 1)

    out = pl.pallas_call(
        functools.partial(_conv_kernel, C=C, Ho=Ho, Wo=Wo),
        out_shape=jax.ShapeDtypeStruct((N, C, Ho * Wo), x.dtype),
        grid=(N,),
        in_specs=[
            pl.BlockSpec((1, C, H * W), lambda n: (n, 0, 0)),
            pl.BlockSpec((9, C, C), lambda n: (0, 0, 0)),
            pl.BlockSpec((C, 1), lambda n: (0, 0)),
        ],
        out_specs=pl.BlockSpec((1, C, Ho * Wo), lambda n: (n, 0, 0)),
        compiler_params=pltpu.CompilerParams(
            dimension_semantics=("parallel",),
            vmem_limit_bytes=_VMEM_LIMIT,
        ),
    )(xf, wT, b2)
    return out.reshape(N, C, Ho, Wo)


# speedup vs baseline: 2.4511x; 2.4511x over previous
import functools

import jax
import jax.numpy as jnp
from jax.experimental import pallas as pl
from jax.experimental.pallas import tpu as pltpu

_VMEM_LIMIT = 64 * 1024 * 1024


def _conv_kernel(x_ref, w_ref, b_ref, o_ref, *, C, Ho, Wo):
    M = Ho * Wo
    H, W = 2 * Ho, 2 * Wo
    vb = x_ref[0].astype(jnp.bfloat16)
    vT = vb.T

    v4 = vT.reshape(Ho, 2, W, C)
    vh0 = v4[:, 0].reshape(Ho * W, C)
    vh1 = v4[:, 1].reshape(Ho * W, C)

    def wsplit(vh):
        u = pltpu.bitcast(vh, jnp.uint32)
        evf = jax.lax.bitcast_convert_type(u << 16, jnp.float32)
        odf = jax.lax.bitcast_convert_type(
            u & jnp.uint32(0xFFFF0000), jnp.float32)
        return evf.astype(jnp.bfloat16), odf.astype(jnp.bfloat16)

    p00, p01 = wsplit(vh0)
    p10, p11 = wsplit(vh1)
    p = ((p00, p01), (p10, p11))

    row = jax.lax.broadcasted_iota(jnp.int32, (M, 1), 0)
    col0 = (row % Wo) == 0

    rmap = ((1, -1), (0, 0), (1, 0))

    def tap(rh, sr, rw, sc):
        q = p[rh][rw]
        k = (-sr) * Wo + (-sc)
        if k:
            q = jnp.concatenate(
                [jnp.zeros((k, C), q.dtype), q[:M - k]], axis=0)
        if sc:
            q = jnp.where(col0, jnp.bfloat16(0), q)
        return q

    acc = jnp.broadcast_to(b_ref[...], (M, C))
    for kh in range(3):
        rh, sr = rmap[kh]
        for kw in range(3):
            rw, sc = rmap[kw]
            acc = acc + jnp.dot(tap(rh, sr, rw, sc), w_ref[kh * 3 + kw],
                                preferred_element_type=jnp.float32)

    o_ref[0] = acc.T


def kernel(x, weight, bias):
    N, C, H, W = x.shape
    Ho, Wo = H // 2, W // 2
    xf = x.reshape(N, C, H * W)
    w9 = weight.reshape(9, C, C).astype(jnp.bfloat16)
    b2 = bias.astype(jnp.float32).reshape(1, C)

    out = pl.pallas_call(
        functools.partial(_conv_kernel, C=C, Ho=Ho, Wo=Wo),
        out_shape=jax.ShapeDtypeStruct((N, C, Ho * Wo), x.dtype),
        grid=(N,),
        in_specs=[
            pl.BlockSpec((1, C, H * W), lambda n: (n, 0, 0)),
            pl.BlockSpec((9, C, C), lambda n: (0, 0, 0)),
            pl.BlockSpec((1, C), lambda n: (0, 0)),
        ],
        out_specs=pl.BlockSpec((1, C, Ho * Wo), lambda n: (n, 0, 0)),
        compiler_params=pltpu.CompilerParams(
            dimension_semantics=("parallel",),
            vmem_limit_bytes=_VMEM_LIMIT,
        ),
    )(xf, w9, b2)
    return out.reshape(N, C, Ho, Wo)
